# Initial kernel scaffold; baseline (speedup 1.0000x reference)
import jax
import jax.numpy as jnp
from jax.experimental import pallas as pl


def kernel(action, pre_embed, W0, W1, W2, W3, W4, bn_gamma, bn_beta):
    raise NotImplementedError("write your pallas kernel here")


# speedup vs baseline: 1.3688x; 1.3688x over previous
import functools

import jax
import jax.numpy as jnp
from jax import lax
from jax.experimental import pallas as pl
from jax.experimental.pallas import tpu as pltpu
from jax.experimental.pallas import tpu_sc as plsc

B = 16384
V = 100000
D = 64
BASE_DIM = D * 5 + D // 2

NW = 32
ROWS_PER_W = B // NW
SUB = 128
NSUB = ROWS_PER_W // SUB

_GATHERS = (
    (0, 0, 0, D),
    (1, 1, D, D),
    (2, 2, 2 * D, D),
    (3, 3, 3 * D, D // 2),
    (4, 4, 3 * D + D // 2, D),
    (5, 1, 4 * D + D // 2, D),
)


def _sc_gather(action_r, W0, W1, W2, W3, W4, pre_embed):
    mesh = plsc.VectorSubcoreMesh(core_axis_name="c", subcore_axis_name="s")

    @functools.partial(
        pl.kernel,
        mesh=mesh,
        out_type=(
            jax.ShapeDtypeStruct((B, BASE_DIM), jnp.float32),
            jax.ShapeDtypeStruct((B, D), jnp.float32),
        ),
        scratch_types=[
            pltpu.VMEM((5, NSUB, SUB), jnp.int32),
            pltpu.VMEM((SUB, D), jnp.float32),
            pltpu.VMEM((SUB, D // 2), jnp.float32),
            pltpu.SemaphoreType.DMA,
        ],
        compiler_params=pltpu.CompilerParams(use_tc_tiling_on_sc=False),
    )
    def k(action_hbm, w0, w1, w2, w3, w4, pre, out_hbm, bias_hbm,
          idx_v, rows64, rows32, sem):
        tables = (w0, w1, w2, w3, w4, pre)
        wid = lax.axis_index("s") * 2 + lax.axis_index("c")
        for c in range(5):
            pltpu.sync_copy(action_hbm.at[c, wid], idx_v.at[c])
        for j in range(NSUB):
            base = wid * ROWS_PER_W + j * SUB
            for (t, c, coff, width) in _GATHERS:
                rbuf = rows64 if width == D else rows32
                pltpu.async_copy(tables[t].at[idx_v.at[c, j]], rbuf, sem).wait()
                pltpu.sync_copy(
                    rbuf, out_hbm.at[pl.ds(base, SUB), pl.ds(coff, width)])
                if t == 2:
                    pltpu.sync_copy(rbuf, bias_hbm.at[pl.ds(base, SUB), :])

    return k(action_r, W0, W1, W2, W3, W4, pre_embed)


_BN_BLOCK = 128
_BN_GRID = (BASE_DIM + _BN_BLOCK - 1) // _BN_BLOCK


def _bn_body(x_ref, g_ref, b_ref, o_ref):
    x = x_ref[...]
    xb = x.astype(jnp.bfloat16).astype(jnp.float32)
    m = jnp.mean(xb, axis=0, keepdims=True)
    d = xb - m
    v = jnp.mean(d * d, axis=0, keepdims=True)
    inv = lax.rsqrt(v + 1e-5)
    o_ref[...] = (d * inv) * g_ref[...] + b_ref[...]


def _tc_batchnorm(x, gamma, beta):
    g2 = gamma.reshape(1, BASE_DIM)
    b2 = beta.reshape(1, BASE_DIM)
    return pl.pallas_call(
        _bn_body,
        grid=(_BN_GRID,),
        in_specs=[
            pl.BlockSpec((B, _BN_BLOCK), lambda i: (0, i)),
            pl.BlockSpec((1, _BN_BLOCK), lambda i: (0, i)),
            pl.BlockSpec((1, _BN_BLOCK), lambda i: (0, i)),
        ],
        out_specs=pl.BlockSpec((B, _BN_BLOCK), lambda i: (0, i)),
        out_shape=jax.ShapeDtypeStruct((B, BASE_DIM), jnp.float32),
    )(x, g2, b2)


def kernel(action, pre_embed, W0, W1, W2, W3, W4, bn_gamma, bn_beta):
    action_r = action.T.reshape(5, NW, NSUB, SUB)
    concat, bias = _sc_gather(action_r, W0, W1, W2, W3, W4, pre_embed)
    embed_base = _tc_batchnorm(concat, bn_gamma, bn_beta)
    return (embed_base, bias)


# speedup vs baseline: 1.3707x; 1.0013x over previous
import functools

import jax
import jax.numpy as jnp
from jax import lax
from jax.experimental import pallas as pl
from jax.experimental.pallas import tpu as pltpu
from jax.experimental.pallas import tpu_sc as plsc

B = 16384
V = 100000
D = 64
BASE_DIM = D * 5 + D // 2

NW = 32
CHUNK = 2048
NCHUNK = B // CHUNK

_TABLES = (
    (0, 0, 0, 2),
    (1, 1, D, 2),
    (2, 2, 2 * D, 2),
    (3, 3, 3 * D, 1),
    (4, 4, 3 * D + D // 2, 2),
    (5, 1, 4 * D + D // 2, 2),
)


def _sc_gather(actT, t0, t1, t2, t3, t4, t5):
    mesh = plsc.VectorSubcoreMesh(core_axis_name="c", subcore_axis_name="s")

    @functools.partial(
        pl.kernel,
        mesh=mesh,
        out_type=jax.ShapeDtypeStruct((BASE_DIM * B,), jnp.float32),
        scratch_types=[
            pltpu.VMEM((V,), jnp.float32),
            pltpu.VMEM((B,), jnp.int32),
            pltpu.VMEM((CHUNK,), jnp.float32),
        ],
        compiler_params=pltpu.CompilerParams(
            use_tc_tiling_on_sc=False, needs_layout_passes=False),
    )
    def k(act_hbm, w0, w1, w2, w3, w4, w5, out_hbm, col_v, idx_v, chunk_v):
        tables = (w0, w1, w2, w3, w4, w5)
        wid = lax.axis_index("s") * 2 + lax.axis_index("c")
        for (t, acol, toff, ncpw) in _TABLES:
            pltpu.sync_copy(act_hbm.at[acol, :], idx_v)
            for kk in range(ncpw):
                d = wid * ncpw + kk
                g = toff + d
                pltpu.sync_copy(tables[t].at[d, :], col_v)
                for ci in range(NCHUNK):

                    def body(j, carry):
                        src = idx_v[pl.ds(ci * CHUNK + j * 16, 16)]
                        chunk_v[pl.ds(j * 16, 16)] = plsc.load_gather(
                            col_v, [src])
                        return carry

                    lax.fori_loop(0, CHUNK // 16, body, 0, unroll=8)
                    pltpu.sync_copy(
                        chunk_v, out_hbm.at[pl.ds(g * B + ci * CHUNK, CHUNK)])

    return k(actT, t0, t1, t2, t3, t4, t5)


_FB = 32
_NFB = BASE_DIM // _FB


def _bn_body(x_ref, g_ref, b_ref, o_ref):
    x = x_ref[...]
    xb = x.astype(jnp.bfloat16).astype(jnp.float32)
    m = jnp.mean(xb, axis=(1, 2), keepdims=True)
    dlt = xb - m
    v = jnp.mean(dlt * dlt, axis=(1, 2), keepdims=True)
    inv = lax.rsqrt(v + 1e-5)
    gg = g_ref[...].reshape(_FB, 1, 1)
    bb = b_ref[...].reshape(_FB, 1, 1)
    o_ref[...] = (dlt * (inv * gg) + bb).reshape(_FB, B)


def _tc_batchnorm(x3, gamma, beta):
    g2 = gamma.reshape(BASE_DIM, 1)
    b2 = beta.reshape(BASE_DIM, 1)
    return pl.pallas_call(
        _bn_body,
        grid=(_NFB,),
        in_specs=[
            pl.BlockSpec((_FB, B // 128, 128), lambda i: (i, 0, 0)),
            pl.BlockSpec((_FB, 1), lambda i: (i, 0)),
            pl.BlockSpec((_FB, 1), lambda i: (i, 0)),
        ],
        out_specs=pl.BlockSpec((_FB, B), lambda i: (i, 0)),
        out_shape=jax.ShapeDtypeStruct((BASE_DIM, B), jnp.float32),
    )(x3, g2, b2)


def kernel(action, pre_embed, W0, W1, W2, W3, W4, bn_gamma, bn_beta):
    actT = action.T
    x1d = _sc_gather(actT, W0.T, W1.T, W2.T, W3.T, W4.T, pre_embed.T)
    x3 = x1d.reshape(BASE_DIM, B // 128, 128)
    yT = _tc_batchnorm(x3, bn_gamma, bn_beta)
    embed_base = yT.T
    embed_bias = x1d[2 * D * B:3 * D * B].reshape(D, B).T
    return (embed_base, embed_bias)


# speedup vs baseline: 2.0055x; 1.4631x over previous
import functools

import jax
import jax.numpy as jnp
from jax import lax
from jax.experimental import pallas as pl
from jax.experimental.pallas import tpu as pltpu
from jax.experimental.pallas import tpu_sc as plsc

B = 16384
V = 100000
D = 64
BASE_DIM = D * 5 + D // 2

NW = 32
CHUNK = 2048
NCHUNK = B // CHUNK
FB = 32

_TABLES = (
    (0, 0, D),
    (1, D, D),
    (2, 2 * D, D),
    (3, 3 * D, D // 2),
    (4, 3 * D + D // 2, D),
    (1, 4 * D + D // 2, D),
)


def _make_sc_gather(acol, ncols):
    ncpw = ncols // NW
    mesh = plsc.VectorSubcoreMesh(core_axis_name="c", subcore_axis_name="s")

    @functools.partial(
        pl.kernel,
        mesh=mesh,
        out_type=jax.ShapeDtypeStruct((ncols * B,), jnp.float32),
        scratch_types=[
            pltpu.VMEM((V,), jnp.float32),
            pltpu.VMEM((B,), jnp.int32),
            pltpu.VMEM((CHUNK,), jnp.float32),
        ],
        compiler_params=pltpu.CompilerParams(
            use_tc_tiling_on_sc=False, needs_layout_passes=False),
    )
    def k(act_hbm, tbl, out_hbm, col_v, idx_v, chunk_v):
        wid = lax.axis_index("s") * 2 + lax.axis_index("c")
        pltpu.sync_copy(act_hbm.at[acol, :], idx_v)
        for kk in range(ncpw):
            d = wid * ncpw + kk
            pltpu.sync_copy(tbl.at[d, :], col_v)
            for ci in range(NCHUNK):

                def body(j, carry):
                    src = idx_v[pl.ds(ci * CHUNK + j * 16, 16)]
                    chunk_v[pl.ds(j * 16, 16)] = plsc.load_gather(
                        col_v, [src])
                    return carry

                lax.fori_loop(0, CHUNK // 16, body, 0, unroll=8)
                pltpu.sync_copy(
                    chunk_v, out_hbm.at[pl.ds(d * B + ci * CHUNK, CHUNK)])

    return k


def _bn_body(x_ref, g_ref, b_ref, _, y_ref):
    x = x_ref[...]
    xb = x.astype(jnp.bfloat16).astype(jnp.float32)
    m = jnp.mean(xb, axis=(1, 2), keepdims=True)
    dlt = xb - m
    v = jnp.mean(dlt * dlt, axis=(1, 2), keepdims=True)
    inv = lax.rsqrt(v + 1e-5)
    gg = g_ref[...].reshape(FB, 1, 1)
    bb = b_ref[...].reshape(FB, 1, 1)
    y_ref[...] = (dlt * (inv * gg) + bb).reshape(FB, B)


def _bn_bias_body(x_ref, g_ref, b_ref, _, y_ref, bias_ref):
    bias_ref[...] = x_ref[...].reshape(FB, B)
    _bn_body(x_ref, g_ref, b_ref, _, y_ref)


def _tc_batchnorm_step(x1d, gamma, beta, y_prev, foff, ncols, with_bias):
    nblk = ncols // FB
    x3 = x1d.reshape(ncols, B // 128, 128)
    g2 = gamma[foff:foff + ncols].reshape(ncols, 1)
    b2 = beta[foff:foff + ncols].reshape(ncols, 1)
    base = foff // FB
    out_shape = [jax.ShapeDtypeStruct((BASE_DIM, B), jnp.float32)]
    out_specs = [pl.BlockSpec((FB, B), lambda i: (base + i, 0))]
    if with_bias:
        out_shape.append(jax.ShapeDtypeStruct((ncols, B), jnp.float32))
        out_specs.append(pl.BlockSpec((FB, B), lambda i: (i, 0)))
    return pl.pallas_call(
        _bn_bias_body if with_bias else _bn_body,
        grid=(nblk,),
        in_specs=[
            pl.BlockSpec((FB, B // 128, 128), lambda i: (i, 0, 0)),
            pl.BlockSpec((FB, 1), lambda i: (i, 0)),
            pl.BlockSpec((FB, 1), lambda i: (i, 0)),
            pl.BlockSpec(memory_space=pl.ANY),
        ],
        out_specs=out_specs,
        out_shape=out_shape,
        input_output_aliases={3: 0},
    )(x3, g2, b2, y_prev)


def kernel(action, pre_embed, W0, W1, W2, W3, W4, bn_gamma, bn_beta):
    actT = action.T
    tablesT = (W0.T, W1.T, W2.T, W3.T, W4.T, pre_embed.T)
    gathered = []
    for ti, (acol, foff, ncols) in enumerate(_TABLES):
        gathered.append(_make_sc_gather(acol, ncols)(actT, tablesT[ti]))

    y = jnp.empty((BASE_DIM, B), dtype=jnp.float32)
    biasT = None
    for ti, (acol, foff, ncols) in enumerate(_TABLES):
        res = _tc_batchnorm_step(
            gathered[ti], bn_gamma, bn_beta, y, foff, ncols, ti == 2)
        if ti == 2:
            y, biasT = res
        else:
            (y,) = res
    return (y.T, biasT.T)


# speedup vs baseline: 2.0555x; 1.0249x over previous
import functools

import jax
import jax.numpy as jnp
from jax import lax
from jax.experimental import pallas as pl
from jax.experimental.pallas import tpu as pltpu
from jax.experimental.pallas import tpu_sc as plsc

B = 16384
V = 100000
D = 64
BASE_DIM = D * 5 + D // 2

NW = 32
CHUNK = 2048
NCHUNK = B // CHUNK
FB = 32

_TABLES = (
    (0, 0, 0, D),
    (1, 1, D, D),
    (2, 2, 2 * D, D),
    (4, 4, 3 * D + D // 2, D),
    (5, 1, 4 * D + D // 2, D),
    (3, 3, 3 * D, D // 2),
)


def _make_sc_gather(acol, ncols):
    ncpw = ncols // NW
    mesh = plsc.VectorSubcoreMesh(core_axis_name="c", subcore_axis_name="s")

    @functools.partial(
        pl.kernel,
        mesh=mesh,
        out_type=jax.ShapeDtypeStruct((ncols * B,), jnp.float32),
        scratch_types=[
            pltpu.VMEM((V,), jnp.float32),
            pltpu.VMEM((B,), jnp.int32),
            pltpu.VMEM((2, CHUNK), jnp.float32),
            pltpu.SemaphoreType.DMA,
        ],
        compiler_params=pltpu.CompilerParams(
            use_tc_tiling_on_sc=False, needs_layout_passes=False),
    )
    def k(act_hbm, tbl, out_hbm, col_v, idx_v, chunk_v, sem):
        wid = lax.axis_index("s") * 2 + lax.axis_index("c")
        H = V // 2
        hidx = pltpu.async_copy(act_hbm.at[acol, :], idx_v, sem)
        d0 = wid * ncpw
        hc0 = pltpu.async_copy(tbl.at[d0, pl.ds(0, H)], col_v.at[pl.ds(0, H)],
                               sem)
        hc1 = pltpu.async_copy(tbl.at[d0, pl.ds(H, H)], col_v.at[pl.ds(H, H)],
                               sem)
        hidx.wait()
        for kk in range(ncpw):
            d = wid * ncpw + kk
            if kk == 0:
                hc0.wait()
                hc1.wait()
            else:
                h0 = pltpu.async_copy(
                    tbl.at[d, pl.ds(0, H)], col_v.at[pl.ds(0, H)], sem)
                h1 = pltpu.async_copy(
                    tbl.at[d, pl.ds(H, H)], col_v.at[pl.ds(H, H)], sem)
                h0.wait()
                h1.wait()
            handles = []
            for ci in range(NCHUNK):
                b = ci % 2
                if ci >= 2:
                    handles[ci - 2].wait()

                def body(j, carry):
                    src = idx_v[pl.ds(ci * CHUNK + j * 16, 16)]
                    chunk_v[b, pl.ds(j * 16, 16)] = plsc.load_gather(
                        col_v, [src])
                    return carry

                lax.fori_loop(0, CHUNK // 16, body, 0, unroll=16)
                handles.append(pltpu.async_copy(
                    chunk_v.at[b],
                    out_hbm.at[pl.ds(d * B + ci * CHUNK, CHUNK)], sem))
            handles[-2].wait()
            handles[-1].wait()

    return k


def _bn_compute(x_ref, g_ref, b_ref, y_ref):
    x = x_ref[...]
    xb = x.astype(jnp.bfloat16).astype(jnp.float32)
    m = jnp.mean(xb, axis=(1, 2), keepdims=True)
    dlt = xb - m
    v = jnp.mean(dlt * dlt, axis=(1, 2), keepdims=True)
    inv = lax.rsqrt(v + 1e-5)
    gg = g_ref[...].reshape(FB, 1, 1)
    bb = b_ref[...].reshape(FB, 1, 1)
    y_ref[...] = (dlt * (inv * gg) + bb).reshape(FB, B)


def _bn_body_first(x_ref, g_ref, b_ref, y_ref):
    _bn_compute(x_ref, g_ref, b_ref, y_ref)


def _bn_body(x_ref, g_ref, b_ref, _, y_ref):
    _bn_compute(x_ref, g_ref, b_ref, y_ref)


def _bn_bias_body(x_ref, g_ref, b_ref, _, y_ref, bias_ref):
    bias_ref[...] = x_ref[...].reshape(FB, B)
    _bn_compute(x_ref, g_ref, b_ref, y_ref)


def _tc_batchnorm_step(x1d, gamma, beta, y_prev, foff, ncols, with_bias):
    nblk = ncols // FB
    x3 = x1d.reshape(ncols, B // 128, 128)
    g2 = gamma[foff:foff + ncols].reshape(ncols, 1)
    b2 = beta[foff:foff + ncols].reshape(ncols, 1)
    base = foff // FB
    out_shape = [jax.ShapeDtypeStruct((BASE_DIM, B), jnp.float32)]
    out_specs = [pl.BlockSpec((FB, B), lambda i: (base + i, 0))]
    if with_bias:
        out_shape.append(jax.ShapeDtypeStruct((ncols, B), jnp.float32))
        out_specs.append(pl.BlockSpec((FB, B), lambda i: (i, 0)))
    in_specs = [
        pl.BlockSpec((FB, B // 128, 128), lambda i: (i, 0, 0)),
        pl.BlockSpec((FB, 1), lambda i: (i, 0)),
        pl.BlockSpec((FB, 1), lambda i: (i, 0)),
    ]
    args = [x3, g2, b2]
    aliases = {}
    if y_prev is not None:
        in_specs.append(pl.BlockSpec(memory_space=pl.ANY))
        args.append(y_prev)
        aliases = {3: 0}
        body = _bn_bias_body if with_bias else _bn_body
    else:
        body = _bn_body_first
    return pl.pallas_call(
        body,
        grid=(nblk,),
        in_specs=in_specs,
        out_specs=out_specs,
        out_shape=out_shape,
        input_output_aliases=aliases,
    )(*args)


def kernel(action, pre_embed, W0, W1, W2, W3, W4, bn_gamma, bn_beta):
    actT = action.T
    tablesT = (W0.T, W1.T, W2.T, W3.T, W4.T, pre_embed.T)
    gathered = {}
    for (ti, acol, foff, ncols) in _TABLES:
        gathered[ti] = _make_sc_gather(acol, ncols)(actT, tablesT[ti])

    y = None
    biasT = None
    for (ti, acol, foff, ncols) in _TABLES:
        res = _tc_batchnorm_step(
            gathered[ti], bn_gamma, bn_beta, y, foff, ncols, ti == 2)
        if ti == 2:
            y, biasT = res
        else:
            (y,) = res
    return (y.T, biasT.T)
